# baseline (speedup 1.0000x reference)
import jax
import jax.numpy as jnp
from jax import lax
from jax.experimental import pallas as pl
from jax.experimental.pallas import tpu as pltpu
from jax.experimental.pallas import tpu_sc as plsc

D_DIM = 32
NI = 4096
NJ = 200
NUM_CORES = 2
NUM_SUBCORES = 16
JG = 8
IG = 4
JPW = NJ // JG
IPW = NI // IG
CH = 512
HALVES = IPW // CH
ITEMS = JPW * HALVES
KT = D_DIM // 8
ITL = CH // 128
SROW = 129
ITLP = ITL + 1
TPR = 4


def _emb_body(w_hbm, idx_hbm, out_hbm, idx_v, g_v, s_v, gsem, ssem):
    c = lax.axis_index("c")
    s = lax.axis_index("s")
    w = s * NUM_CORES + c
    jg = w // IG
    ig = w % IG
    j0 = jg * JPW
    it_base = ig * (IPW // 128)

    pltpu.sync_copy(idx_hbm.at[pl.ds(j0, JPW), pl.ds(ig * IPW, IPW)], idx_v)

    iota = lax.iota(jnp.int32, 16)
    ktv0 = iota // 8
    krv0 = iota % 8
    ktv1 = (iota + 16) // 8
    krv1 = (iota + 16) % 8
    zerov = iota - iota
    rows0 = [ktv0 * (ITLP * 8) + itl * 8 + krv0 for itl in range(ITL)]
    rows1 = [ktv1 * (ITLP * 8) + itl * 8 + krv1 for itl in range(ITL)]

    def start_item(g, b):
        jl = g // HALVES
        half = g % HALVES
        pltpu.async_copy(w_hbm.at[idx_v.at[jl, pl.ds(half * CH, CH)]],
                         g_v.at[b], gsem.at[b])

    def wait_item(g, b):
        jl = g // HALVES
        half = g % HALVES
        pltpu.make_async_copy(w_hbm.at[idx_v.at[jl, pl.ds(half * CH, CH)]],
                              g_v.at[b], gsem.at[b]).wait()

    def transpose(b):
        gv = g_v.at[b]
        sv = s_v.at[b]
        for itl in range(ITL):
            r0v = rows0[itl]
            r1v = rows1[itl]

            def ilbody(z, _, r0v=r0v, r1v=r1v, itl=itl):
                for u in range(8):
                    il = z * 8 + u
                    r = itl * 128 + il
                    ilv = zerov + il
                    plsc.store_scatter(sv, [r0v, ilv],
                                       gv[r, pl.ds(0, 16)])
                    plsc.store_scatter(sv, [r1v, ilv],
                                       gv[r, pl.ds(16, 16)])
                return 0

            lax.fori_loop(0, 16, ilbody, 0)

    def store_start(g, b):
        jl = g // HALVES
        half = g % HALVES
        for kt in range(KT):
            for itl in range(ITL):
                pltpu.async_copy(
                    s_v.at[b, pl.ds((kt * ITLP + itl) * 8, 8), pl.ds(0, 128)],
                    out_hbm.at[j0 + jl, kt, it_base + half * ITL + itl],
                    ssem.at[b])

    def store_wait(g, b):
        jl = g // HALVES
        half = g % HALVES
        for kt in range(KT):
            for itl in range(ITL):
                pltpu.make_async_copy(
                    s_v.at[b, pl.ds((kt * ITLP + itl) * 8, 8), pl.ds(0, 128)],
                    out_hbm.at[j0 + jl, kt, it_base + half * ITL + itl],
                    ssem.at[b]).wait()

    start_item(0, 0)
    start_item(1, 1)

    def outer(t, _):
        for par, b in ((0, 0), (1, 1)):
            g = 2 * t + par
            wait_item(g, b)

            @pl.when(t >= 1)
            def _(g=g, b=b):
                store_wait(g - 2, b)

            transpose(b)
            store_start(g, b)

            @pl.when(t <= ITEMS // 2 - 2)
            def _(g=g, b=b):
                start_item(g + 2, b)
        return 0

    lax.fori_loop(0, ITEMS // 2, outer, 0)
    store_wait(ITEMS - 2, 0)
    store_wait(ITEMS - 1, 1)


def kernel(weight, token_ids):
    idx_t = token_ids.T.astype(jnp.int32)
    mesh = plsc.VectorSubcoreMesh(core_axis_name="c", subcore_axis_name="s")
    out_p = pl.kernel(
        _emb_body,
        out_type=jax.ShapeDtypeStruct((NJ, KT, NI // 128, 8, 128),
                                      jnp.float32),
        mesh=mesh,
        scratch_types=[
            pltpu.VMEM((JPW, IPW), jnp.int32),
            pltpu.VMEM((2, CH, D_DIM), jnp.float32),
            pltpu.VMEM((2, KT * ITLP * 8, SROW), jnp.float32),
            pltpu.SemaphoreType.DMA((2,)),
            pltpu.SemaphoreType.DMA((2,)),
        ],
        compiler_params=pltpu.CompilerParams(use_tc_tiling_on_sc=False,
                                             needs_layout_passes=False),
    )(weight, idx_t)
    return out_p.transpose(2, 4, 0, 1, 3).reshape(NI, NJ, D_DIM)
